# baseline (speedup 1.0000x reference)
import functools

import jax
import jax.numpy as jnp
from jax import lax
from jax.experimental import pallas as pl
from jax.experimental.pallas import tpu as pltpu
from jax.experimental.pallas import tpu_sc as plsc

F32 = jnp.float32

RBF_DMAX = 15.0
RBF_DIM = 16
MSG_NORM = 10.0

NUM_CORES = 2
NUM_SUBCORES = 16
NW = NUM_CORES * NUM_SUBCORES
CH = 128

_EDGE_BLOCK = 1024
_NODE_BLOCK = 2048


def _sc_gather(table, coord4, src_w, dst_w):
    nch = src_w.shape[1]
    d = table.shape[1]
    node_pad = coord4.shape[0] // 4
    e_pad = NW * nch * CH
    mesh = plsc.VectorSubcoreMesh(core_axis_name="c", subcore_axis_name="s")

    @functools.partial(
        pl.kernel,
        mesh=mesh,
        out_type=[
            jax.ShapeDtypeStruct((e_pad, d), F32),
            jax.ShapeDtypeStruct((e_pad, 3), F32),
        ],
        scratch_types=[
            pltpu.VMEM((nch, CH), jnp.int32),
            pltpu.VMEM((nch, CH), jnp.int32),
            pltpu.VMEM((2, CH, d), F32),
            pltpu.VMEM((2, CH, 3), F32),
            pltpu.VMEM((node_pad * 4,), F32),
            pltpu.SemaphoreType.DMA,
            pltpu.SemaphoreType.DMA,
            pltpu.SemaphoreType.DMA,
            pltpu.SemaphoreType.DMA,
        ],
        compiler_params=pltpu.CompilerParams(use_tc_tiling_on_sc=False,
                                             needs_layout_passes=False),
    )
    def gk(table_hbm, coord_hbm, src_hbm, dst_hbm,
           rows_out, diff_out,
           idx_s, idx_d, rows_v, diff_v, ctab,
           sem0, sem1, sem2, sem3):
        wid = lax.axis_index("s") * NUM_CORES + lax.axis_index("c")
        base = wid * (nch * CH)
        pltpu.sync_copy(src_hbm.at[wid], idx_s)
        pltpu.sync_copy(dst_hbm.at[wid], idx_d)
        pltpu.sync_copy(coord_hbm, ctab)
        gsem = [sem0, sem1]

        def gstart(slot, j):
            pltpu.async_copy(table_hbm.at[idx_s.at[j]], rows_v.at[slot],
                             gsem[slot])

        def gwait(slot, j):
            pltpu.make_async_copy(table_hbm.at[idx_s.at[j]],
                                  rows_v.at[slot], gsem[slot]).wait()

        def coords(slot, j):
            for t in range(CH // 16):
                svec = idx_s[j, pl.ds(t * 16, 16)] * 4
                dvec = idx_d[j, pl.ds(t * 16, 16)] * 4
                rowv = lax.iota(jnp.int32, 16) + (t * 16)
                for c in range(3):
                    gs = plsc.load_gather(ctab, [svec + c])
                    gd = plsc.load_gather(ctab, [dvec + c])
                    colv = jnp.full((16,), c, jnp.int32)
                    plsc.store_scatter(diff_v.at[slot], [rowv, colv],
                                       gs - gd)

        def store(slot, j):
            off = base + j * CH
            c1 = pltpu.async_copy(rows_v.at[slot],
                                  rows_out.at[pl.ds(off, CH)], sem2)
            c2 = pltpu.async_copy(diff_v.at[slot],
                                  diff_out.at[pl.ds(off, CH)], sem3)
            c1.wait()
            c2.wait()

        gstart(0, 0)

        def body(jj, carry):
            j0 = jj * 2
            j1 = j0 + 1
            gstart(1, j1)
            coords(0, j0)
            gwait(0, j0)
            store(0, j0)

            @pl.when(j0 + 2 < nch)
            def _():
                gstart(0, j0 + 2)

            coords(1, j1)
            gwait(1, j1)
            store(1, j1)
            return carry

        lax.fori_loop(0, nch // 2, body, 0)

    return gk(table, coord4, src_w, dst_w)


def _sc_scatter(msg2, dst_t, zeros_hbm):
    nch2 = dst_t.shape[1]
    dm = msg2.shape[2]
    node_pad = zeros_hbm.shape[0]
    rows_per = node_pad // NUM_SUBCORES
    mesh = plsc.VectorSubcoreMesh(core_axis_name="c", subcore_axis_name="s")

    @functools.partial(
        pl.kernel,
        mesh=mesh,
        out_type=jax.ShapeDtypeStruct((NUM_CORES, node_pad, dm), F32),
        scratch_types=[
            pltpu.VMEM((nch2, CH), jnp.int32),
            pltpu.VMEM((2, CH, dm), F32),
            pltpu.VMEM_SHARED((node_pad, dm), F32),
            pltpu.SemaphoreType.DMA,
            pltpu.SemaphoreType.DMA,
        ],
    )
    def sk(msg_hbm, dst_hbm, zero_hbm, out_hbm, idx_d, msg_v, acc,
           sem0, sem1):
        cid = lax.axis_index("c")
        sid = lax.axis_index("s")
        lsem = [sem0, sem1]

        pltpu.sync_copy(zero_hbm.at[pl.ds(sid * rows_per, rows_per)],
                        acc.at[pl.ds(sid * rows_per, rows_per)])
        plsc.subcore_barrier()

        pltpu.sync_copy(dst_hbm.at[sid], idx_d)

        def lstart(slot, j):
            off = (sid * nch2 + j) * CH
            pltpu.async_copy(msg_hbm.at[cid].at[pl.ds(off, CH)],
                             msg_v.at[slot], lsem[slot])

        def lwait(slot, j):
            off = (sid * nch2 + j) * CH
            pltpu.make_async_copy(msg_hbm.at[cid].at[pl.ds(off, CH)],
                                  msg_v.at[slot], lsem[slot]).wait()

        def accum(slot, j):
            pltpu.sync_copy(msg_v.at[slot], acc.at[idx_d.at[j]], add=True)

        lstart(0, 0)

        def body(jj, carry):
            j0 = jj * 2
            j1 = j0 + 1
            lstart(1, j1)
            lwait(0, j0)
            accum(0, j0)

            @pl.when(jj + 1 < nch2 // 2)
            def _():
                lstart(0, j0 + 2)

            lwait(1, j1)
            accum(1, j1)
            return carry

        lax.fori_loop(0, nch2 // 2, body, 0)
        plsc.subcore_barrier()

        pltpu.sync_copy(acc.at[pl.ds(sid * rows_per, rows_per)],
                        out_hbm.at[cid].at[pl.ds(sid * rows_per, rows_per)])

    return sk(msg2, dst_t, zeros_hbm)


def _edge_tc(rows, diff_in, msg_params, s, v):
    e_pad, d = rows.shape
    wh, wu, wm, bm, wg, bg = msg_params
    h = wh.shape[1]
    wh0 = wh[0:1]
    wh1 = wh[1:]
    wms = wm[:s]
    wmd = wm[s:s + RBF_DIM]
    wmh = wm[s + RBF_DIM:]
    bm2 = bm.reshape(1, -1)
    bg2 = bg.reshape(1, -1)

    def body(rows_ref, diff_ref, wh0_ref, wh1_ref, wu_ref,
             wms_ref, wmd_ref, wmh_ref, bm_ref, wg_ref, bg_ref, out_ref):
        dot = functools.partial(jnp.dot, preferred_element_type=F32)
        rows_b = rows_ref[...]
        sca = rows_b[:, :s]
        diff = diff_ref[...]
        d2 = jnp.sum(diff * diff, axis=1, keepdims=True)
        dij = jnp.sqrt(jnp.maximum(d2, 1e-8)) + 1e-8
        diffn = diff / dij
        sigma = RBF_DMAX / RBF_DIM
        mu = lax.broadcasted_iota(jnp.int32, (1, RBF_DIM), 1).astype(F32) * (
            RBF_DMAX / (RBF_DIM - 1))
        z = (dij - mu) / sigma
        d_rbf = jnp.exp(-(z * z))

        wh0_b = wh0_ref[...]
        wh1_b = wh1_ref[...]
        wu_b = wu_ref[...]
        vu = []
        sh2 = None
        for c in range(3):
            xc = diffn[:, c:c + 1]
            vc = rows_b[:, s + v * c:s + v * (c + 1)]
            vh_c = xc * wh0_b + dot(vc, wh1_b)
            vu.append(dot(vh_c, wu_b))
            sq = vh_c * vh_c
            sh2 = sq if sh2 is None else sh2 + sq
        sh = jnp.sqrt(jnp.maximum(sh2, 1e-8))
        pre = (dot(sca, wms_ref[...]) + dot(d_rbf, wmd_ref[...])
               + dot(sh, wmh_ref[...]) + bm_ref[...])
        feats = pre * jax.nn.sigmoid(pre)
        gate = jax.nn.sigmoid(dot(feats, wg_ref[...]) + bg_ref[...])
        out_ref[0] = feats
        out_ref[1] = jnp.concatenate(
            [gate * vu[c] for c in range(3)]
            + [jnp.zeros((gate.shape[0], 80), F32)], axis=1)

    b = _EDGE_BLOCK
    grid = (e_pad // b,)
    full = lambda a: pl.BlockSpec(a.shape, lambda i: (0,) * a.ndim)
    consts = (wh0, wh1, wu, wms, wmd, wmh, bm2, wg, bg2)
    return pl.pallas_call(
        body,
        grid=grid,
        in_specs=[
            pl.BlockSpec((b, d), lambda i: (i, 0)),
            pl.BlockSpec((b, 3), lambda i: (i, 0)),
        ] + [full(c) for c in consts],
        out_specs=pl.BlockSpec((2, b, 128), lambda i: (0, i, 0)),
        out_shape=jax.ShapeDtypeStruct((2, e_pad, 128), F32),
    )(rows, diff_in, *consts)


def _node_tc(sca_pad, vec_pad, parts, upd_params, ln1_g, ln1_b, ln2_g, ln2_b,
             s, v):
    node_pad = sca_pad.shape[0]
    dm = parts.shape[2]
    wh, wu, wm, bm, wg, bg = upd_params
    wms = wm[:s]
    wmh = wm[s:]
    bm2 = bm.reshape(1, -1)
    bg2 = bg.reshape(1, -1)
    g1 = ln1_g.reshape(1, -1)
    b1 = ln1_b.reshape(1, -1)
    g2 = ln2_g.reshape(1, -1)
    b2 = ln2_b.reshape(1, -1)

    def layernorm(sx, v3, g_ref, b_ref):
        mu = jnp.mean(sx, axis=1, keepdims=True)
        xc = sx - mu
        var = jnp.mean(xc * xc, axis=1, keepdims=True)
        s_out = xc / jnp.sqrt(var + 1e-5) * g_ref[...] + b_ref[...]
        vn = jnp.maximum(v3[0] * v3[0] + v3[1] * v3[1] + v3[2] * v3[2], 1e-8)
        vnm = jnp.sqrt(jnp.mean(vn, axis=1, keepdims=True))
        return s_out, [vc / vnm for vc in v3]

    def body(sca_ref, vec_ref, parts_ref, g1_ref, b1_ref, wh_ref, wu_ref,
             wms_ref, wmh_ref, bm_ref, wg_ref, bg_ref, g2_ref, b2_ref,
             s_out_ref, v_out_ref):
        a0 = parts_ref[0]
        a1 = parts_ref[1]
        inv = 1.0 / MSG_NORM
        sx = sca_ref[...] + a0 * inv
        v3 = [vec_ref[:, v * c:v * (c + 1)]
              + a1[:, v * c:v * (c + 1)] * inv for c in range(3)]
        s1, v1 = layernorm(sx, v3, g1_ref, b1_ref)

        wh_b = wh_ref[...]
        wu_b = wu_ref[...]
        vu = []
        sh2 = None
        for c in range(3):
            vh_c = jnp.dot(v1[c], wh_b, preferred_element_type=F32)
            vu.append(jnp.dot(vh_c, wu_b, preferred_element_type=F32))
            sq = vh_c * vh_c
            sh2 = sq if sh2 is None else sh2 + sq
        sh = jnp.sqrt(jnp.maximum(sh2, 1e-8))
        pre = (jnp.dot(s1, wms_ref[...], preferred_element_type=F32)
               + jnp.dot(sh, wmh_ref[...], preferred_element_type=F32)
               + bm_ref[...])
        feats = pre * jax.nn.sigmoid(pre)
        gate = jax.nn.sigmoid(
            jnp.dot(feats, wg_ref[...], preferred_element_type=F32)
            + bg_ref[...])
        s2 = s1 + feats
        v2 = [v1[c] + gate * vu[c] for c in range(3)]
        s3, v3o = layernorm(s2, v2, g2_ref, b2_ref)
        s_out_ref[...] = s3
        for c in range(3):
            v_out_ref[:, v * c:v * (c + 1)] = v3o[c]

    bn = _NODE_BLOCK
    grid = (node_pad // bn,)
    full = lambda a: pl.BlockSpec(a.shape, lambda i: (0,) * a.ndim)
    return pl.pallas_call(
        body,
        grid=grid,
        in_specs=[
            pl.BlockSpec((bn, s), lambda i: (i, 0)),
            pl.BlockSpec((bn, 3 * v), lambda i: (i, 0)),
            pl.BlockSpec((2, bn, dm), lambda i: (0, i, 0)),
            full(g1), full(b1), full(wh), full(wu), full(wms), full(wmh),
            full(bm2), full(wg), full(bg2), full(g2), full(b2),
        ],
        out_specs=[
            pl.BlockSpec((bn, s), lambda i: (i, 0)),
            pl.BlockSpec((bn, 3 * v), lambda i: (i, 0)),
        ],
        out_shape=[
            jax.ShapeDtypeStruct((node_pad, s), F32),
            jax.ShapeDtypeStruct((node_pad, 3 * v), F32),
        ],
    )(sca_pad, vec_pad, parts, g1, b1, wh, wu, wms, wmh, bm2, wg, bg2, g2,
      b2)


def kernel(scalar_feat, coord_feat, vec_feat, edge_index, msg_params,
           upd_params, ln1_g, ln1_b, ln2_g, ln2_b):
    n, s = scalar_feat.shape
    v = vec_feat.shape[1]
    e = edge_index.shape[1]

    nch = -(-e // (NW * CH))
    e_pad = NW * nch * CH
    node_pad = -(-(n + 1) // _NODE_BLOCK) * _NODE_BLOCK

    src = edge_index[0].astype(jnp.int32)
    dst = edge_index[1].astype(jnp.int32)
    src_w = jnp.concatenate(
        [src, jnp.zeros((e_pad - e,), jnp.int32)]).reshape(NW, nch, CH)
    dst_p = jnp.concatenate(
        [dst, jnp.full((e_pad - e,), node_pad - 1, jnp.int32)])
    dst_w = dst_p.reshape(NW, nch, CH)

    vec_cm = vec_feat.transpose(0, 2, 1).reshape(n, 3 * v)
    table = jnp.concatenate([scalar_feat, vec_cm], axis=1)
    coord4 = jnp.pad(coord_feat, ((0, node_pad - n), (0, 1))).reshape(-1)

    rows, diff = _sc_gather(table, coord4, src_w, dst_w)
    msg = _edge_tc(rows, diff, msg_params, s, v)

    nch2 = e_pad // (NUM_SUBCORES * CH)
    dst_t = dst_p.reshape(NUM_SUBCORES, nch2, CH)
    zeros_hbm = jnp.zeros((node_pad, 128), F32)
    parts = _sc_scatter(msg, dst_t, zeros_hbm)

    sca_pad = jnp.pad(scalar_feat, ((0, node_pad - n), (0, 0)))
    vec_pad = jnp.pad(vec_cm, ((0, node_pad - n), (0, 0)))
    s_full, v_full = _node_tc(sca_pad, vec_pad, parts, upd_params,
                              ln1_g, ln1_b, ln2_g, ln2_b, s, v)

    s_out = s_full[:n]
    v_out = v_full[:n].reshape(n, 3, v).transpose(0, 2, 1)
    return (s_out, v_out)


# speedup vs baseline: 1.1005x; 1.1005x over previous
import functools

import jax
import jax.numpy as jnp
from jax import lax
from jax.experimental import pallas as pl
from jax.experimental.pallas import tpu as pltpu
from jax.experimental.pallas import tpu_sc as plsc

F32 = jnp.float32

RBF_DMAX = 15.0
RBF_DIM = 16
MSG_NORM = 10.0

NUM_CORES = 2
NUM_SUBCORES = 16
NW = NUM_CORES * NUM_SUBCORES
CH = 128

_EDGE_BLOCK = 1024
_NODE_BLOCK = 2048


def _sc_gather(table, aux, coordt, src_w, dst_w):
    nch = src_w.shape[1]
    d = table.shape[1]
    da = aux.shape[1]
    e_pad = NW * nch * CH
    mesh = plsc.VectorSubcoreMesh(core_axis_name="c", subcore_axis_name="s")

    @functools.partial(
        pl.kernel,
        mesh=mesh,
        out_type=[
            jax.ShapeDtypeStruct((e_pad, d), F32),
            jax.ShapeDtypeStruct((e_pad, da), F32),
            jax.ShapeDtypeStruct((e_pad, 16), F32),
        ],
        scratch_types=[
            pltpu.VMEM((nch, CH), jnp.int32),
            pltpu.VMEM((nch, CH), jnp.int32),
            pltpu.VMEM((4, CH, d), F32),
            pltpu.VMEM((4, CH, da), F32),
            pltpu.VMEM((4, CH, 16), F32),
            pltpu.SemaphoreType.DMA,
            pltpu.SemaphoreType.DMA,
            pltpu.SemaphoreType.DMA,
            pltpu.SemaphoreType.DMA,
            pltpu.SemaphoreType.DMA,
            pltpu.SemaphoreType.DMA,
        ],
        compiler_params=pltpu.CompilerParams(use_tc_tiling_on_sc=False),
    )
    def gk(table_hbm, aux_hbm, coord_hbm, src_hbm, dst_hbm,
           rows_out, aux_out, cd_out,
           idx_s, idx_d, rows_v, aux_v, cd_v,
           sem0, sem1, sem2, sem3, sem4, sem5):
        wid = lax.axis_index("s") * NUM_CORES + lax.axis_index("c")
        base = wid * (nch * CH)
        pltpu.sync_copy(src_hbm.at[wid], idx_s)
        pltpu.sync_copy(dst_hbm.at[wid], idx_d)
        gsem = [sem0, sem1, sem4, sem5]

        def gstart(slot, j):
            pltpu.async_copy(table_hbm.at[idx_s.at[j]], rows_v.at[slot],
                             gsem[slot])
            pltpu.async_copy(aux_hbm.at[idx_s.at[j]], aux_v.at[slot],
                             gsem[slot])
            pltpu.async_copy(coord_hbm.at[idx_d.at[j]], cd_v.at[slot],
                             gsem[slot])

        def gwait(slot, j):
            pltpu.make_async_copy(table_hbm.at[idx_s.at[j]],
                                  rows_v.at[slot], gsem[slot]).wait()
            pltpu.make_async_copy(aux_hbm.at[idx_s.at[j]],
                                  aux_v.at[slot], gsem[slot]).wait()
            pltpu.make_async_copy(coord_hbm.at[idx_d.at[j]],
                                  cd_v.at[slot], gsem[slot]).wait()

        def store(slot, j):
            off = base + j * CH
            c1 = pltpu.async_copy(rows_v.at[slot],
                                  rows_out.at[pl.ds(off, CH)], sem2)
            c2 = pltpu.async_copy(aux_v.at[slot],
                                  aux_out.at[pl.ds(off, CH)], sem2)
            c3 = pltpu.async_copy(cd_v.at[slot],
                                  cd_out.at[pl.ds(off, CH)], sem3)
            c1.wait()
            c2.wait()
            c3.wait()

        nslots = 4
        for k in range(nslots):
            gstart(k, k)

        def body(jj, carry):
            jbase = jj * nslots
            for k in range(nslots):
                j = jbase + k
                gwait(k, j)
                store(k, j)

                @pl.when(j + nslots < nch)
                def _():
                    gstart(k, j + nslots)

            return carry

        lax.fori_loop(0, nch // nslots, body, 0)

    return gk(table, aux, coordt, src_w, dst_w)


def _sc_scatter(msg2, dst_t, zeros_hbm):
    nch2 = dst_t.shape[1]
    dm = msg2.shape[2]
    node_pad = zeros_hbm.shape[0]
    rows_per = node_pad // NUM_SUBCORES
    mesh = plsc.VectorSubcoreMesh(core_axis_name="c", subcore_axis_name="s")

    @functools.partial(
        pl.kernel,
        mesh=mesh,
        out_type=jax.ShapeDtypeStruct((NUM_CORES, node_pad, dm), F32),
        scratch_types=[
            pltpu.VMEM((nch2, CH), jnp.int32),
            pltpu.VMEM((2, CH, dm), F32),
            pltpu.VMEM_SHARED((node_pad, dm), F32),
            pltpu.SemaphoreType.DMA,
            pltpu.SemaphoreType.DMA,
        ],
    )
    def sk(msg_hbm, dst_hbm, zero_hbm, out_hbm, idx_d, msg_v, acc,
           sem0, sem1):
        cid = lax.axis_index("c")
        sid = lax.axis_index("s")
        lsem = [sem0, sem1]

        pltpu.sync_copy(zero_hbm.at[pl.ds(sid * rows_per, rows_per)],
                        acc.at[pl.ds(sid * rows_per, rows_per)])
        plsc.subcore_barrier()

        pltpu.sync_copy(dst_hbm.at[sid], idx_d)

        def lstart(slot, j):
            off = (sid * nch2 + j) * CH
            pltpu.async_copy(msg_hbm.at[cid].at[pl.ds(off, CH)],
                             msg_v.at[slot], lsem[slot])

        def lwait(slot, j):
            off = (sid * nch2 + j) * CH
            pltpu.make_async_copy(msg_hbm.at[cid].at[pl.ds(off, CH)],
                                  msg_v.at[slot], lsem[slot]).wait()

        def accum(slot, j):
            pltpu.sync_copy(msg_v.at[slot], acc.at[idx_d.at[j]], add=True)

        lstart(0, 0)

        def body(jj, carry):
            j0 = jj * 2
            j1 = j0 + 1
            lstart(1, j1)
            lwait(0, j0)
            accum(0, j0)

            @pl.when(jj + 1 < nch2 // 2)
            def _():
                lstart(0, j0 + 2)

            lwait(1, j1)
            accum(1, j1)
            return carry

        lax.fori_loop(0, nch2 // 2, body, 0)
        plsc.subcore_barrier()

        pltpu.sync_copy(acc.at[pl.ds(sid * rows_per, rows_per)],
                        out_hbm.at[cid].at[pl.ds(sid * rows_per, rows_per)])

    return sk(msg2, dst_t, zeros_hbm)


def _edge_tc(rows, aux, coordd, msg_params, s, v):
    e_pad, d = rows.shape
    da = aux.shape[1]
    wh, wu, wm, bm, wg, bg = msg_params
    h = wh.shape[1]
    wh0 = wh[0:1]
    wh1 = wh[1:]
    wms = wm[:s]
    wmd = wm[s:s + RBF_DIM]
    wmh = wm[s + RBF_DIM:]
    bm2 = bm.reshape(1, -1)
    bg2 = bg.reshape(1, -1)

    def body(rows_ref, aux_ref, cd_ref, wh0_ref, wh1_ref, wu_ref,
             wms_ref, wmd_ref, wmh_ref, bm_ref, wg_ref, bg_ref, out_ref):
        dot = functools.partial(jnp.dot, preferred_element_type=F32)
        sca = rows_ref[...]
        aux_b = aux_ref[...]
        xs = aux_b[:, 3 * v:]
        xd = cd_ref[...]
        diff = xs - xd
        d2 = jnp.sum(diff * diff, axis=1, keepdims=True)
        dij = jnp.sqrt(jnp.maximum(d2, 1e-8)) + 1e-8
        diffn = diff / dij
        sigma = RBF_DMAX / RBF_DIM
        mu = lax.broadcasted_iota(jnp.int32, (1, RBF_DIM), 1).astype(F32) * (
            RBF_DMAX / (RBF_DIM - 1))
        z = (dij - mu) / sigma
        d_rbf = jnp.exp(-(z * z))

        wh0_b = wh0_ref[...]
        wh1_b = wh1_ref[...]
        wu_b = wu_ref[...]
        vu = []
        sh2 = None
        for c in range(3):
            xc = diffn[:, c:c + 1]
            vc = aux_b[:, v * c:v * (c + 1)]
            vh_c = xc * wh0_b + dot(vc, wh1_b)
            vu.append(dot(vh_c, wu_b))
            sq = vh_c * vh_c
            sh2 = sq if sh2 is None else sh2 + sq
        sh = jnp.sqrt(jnp.maximum(sh2, 1e-8))
        pre = (dot(sca, wms_ref[...]) + dot(d_rbf, wmd_ref[...])
               + dot(sh, wmh_ref[...]) + bm_ref[...])
        feats = pre * jax.nn.sigmoid(pre)
        gate = jax.nn.sigmoid(dot(feats, wg_ref[...]) + bg_ref[...])
        out_ref[0] = feats
        out_ref[1] = jnp.concatenate(
            [gate * vu[c] for c in range(3)]
            + [jnp.zeros((gate.shape[0], 80), F32)], axis=1)

    b = _EDGE_BLOCK
    grid = (e_pad // b,)
    full = lambda a: pl.BlockSpec(a.shape, lambda i: (0,) * a.ndim)
    consts = (wh0, wh1, wu, wms, wmd, wmh, bm2, wg, bg2)
    return pl.pallas_call(
        body,
        grid=grid,
        in_specs=[
            pl.BlockSpec((b, d), lambda i: (i, 0)),
            pl.BlockSpec((b, da), lambda i: (i, 0)),
            pl.BlockSpec((b, 16), lambda i: (i, 0)),
        ] + [full(c) for c in consts],
        out_specs=pl.BlockSpec((2, b, 128), lambda i: (0, i, 0)),
        out_shape=jax.ShapeDtypeStruct((2, e_pad, 128), F32),
    )(rows, aux, coordd, *consts)


def _node_tc(sca_pad, vec_pad, parts, upd_params, ln1_g, ln1_b, ln2_g, ln2_b,
             s, v):
    node_pad = sca_pad.shape[0]
    dm = parts.shape[2]
    wh, wu, wm, bm, wg, bg = upd_params
    wms = wm[:s]
    wmh = wm[s:]
    bm2 = bm.reshape(1, -1)
    bg2 = bg.reshape(1, -1)
    g1 = ln1_g.reshape(1, -1)
    b1 = ln1_b.reshape(1, -1)
    g2 = ln2_g.reshape(1, -1)
    b2 = ln2_b.reshape(1, -1)

    def layernorm(sx, v3, g_ref, b_ref):
        mu = jnp.mean(sx, axis=1, keepdims=True)
        xc = sx - mu
        var = jnp.mean(xc * xc, axis=1, keepdims=True)
        s_out = xc / jnp.sqrt(var + 1e-5) * g_ref[...] + b_ref[...]
        vn = jnp.maximum(v3[0] * v3[0] + v3[1] * v3[1] + v3[2] * v3[2], 1e-8)
        vnm = jnp.sqrt(jnp.mean(vn, axis=1, keepdims=True))
        return s_out, [vc / vnm for vc in v3]

    def body(sca_ref, vec_ref, parts_ref, g1_ref, b1_ref, wh_ref, wu_ref,
             wms_ref, wmh_ref, bm_ref, wg_ref, bg_ref, g2_ref, b2_ref,
             s_out_ref, v_out_ref):
        a0 = parts_ref[0]
        a1 = parts_ref[1]
        inv = 1.0 / MSG_NORM
        sx = sca_ref[...] + a0 * inv
        v3 = [vec_ref[:, v * c:v * (c + 1)]
              + a1[:, v * c:v * (c + 1)] * inv for c in range(3)]
        s1, v1 = layernorm(sx, v3, g1_ref, b1_ref)

        wh_b = wh_ref[...]
        wu_b = wu_ref[...]
        vu = []
        sh2 = None
        for c in range(3):
            vh_c = jnp.dot(v1[c], wh_b, preferred_element_type=F32)
            vu.append(jnp.dot(vh_c, wu_b, preferred_element_type=F32))
            sq = vh_c * vh_c
            sh2 = sq if sh2 is None else sh2 + sq
        sh = jnp.sqrt(jnp.maximum(sh2, 1e-8))
        pre = (jnp.dot(s1, wms_ref[...], preferred_element_type=F32)
               + jnp.dot(sh, wmh_ref[...], preferred_element_type=F32)
               + bm_ref[...])
        feats = pre * jax.nn.sigmoid(pre)
        gate = jax.nn.sigmoid(
            jnp.dot(feats, wg_ref[...], preferred_element_type=F32)
            + bg_ref[...])
        s2 = s1 + feats
        v2 = [v1[c] + gate * vu[c] for c in range(3)]
        s3, v3o = layernorm(s2, v2, g2_ref, b2_ref)
        s_out_ref[...] = s3
        for c in range(3):
            v_out_ref[:, v * c:v * (c + 1)] = v3o[c]

    bn = _NODE_BLOCK
    grid = (node_pad // bn,)
    full = lambda a: pl.BlockSpec(a.shape, lambda i: (0,) * a.ndim)
    return pl.pallas_call(
        body,
        grid=grid,
        in_specs=[
            pl.BlockSpec((bn, s), lambda i: (i, 0)),
            pl.BlockSpec((bn, 3 * v), lambda i: (i, 0)),
            pl.BlockSpec((2, bn, dm), lambda i: (0, i, 0)),
            full(g1), full(b1), full(wh), full(wu), full(wms), full(wmh),
            full(bm2), full(wg), full(bg2), full(g2), full(b2),
        ],
        out_specs=[
            pl.BlockSpec((bn, s), lambda i: (i, 0)),
            pl.BlockSpec((bn, 3 * v), lambda i: (i, 0)),
        ],
        out_shape=[
            jax.ShapeDtypeStruct((node_pad, s), F32),
            jax.ShapeDtypeStruct((node_pad, 3 * v), F32),
        ],
    )(sca_pad, vec_pad, parts, g1, b1, wh, wu, wms, wmh, bm2, wg, bg2, g2,
      b2)


def kernel(scalar_feat, coord_feat, vec_feat, edge_index, msg_params,
           upd_params, ln1_g, ln1_b, ln2_g, ln2_b):
    n, s = scalar_feat.shape
    v = vec_feat.shape[1]
    e = edge_index.shape[1]

    nch = -(-e // (NW * CH))
    e_pad = NW * nch * CH
    node_pad = -(-(n + 1) // _NODE_BLOCK) * _NODE_BLOCK

    src = edge_index[0].astype(jnp.int32)
    dst = edge_index[1].astype(jnp.int32)
    src_w = jnp.concatenate(
        [src, jnp.zeros((e_pad - e,), jnp.int32)]).reshape(NW, nch, CH)
    dst_p = jnp.concatenate(
        [dst, jnp.full((e_pad - e,), node_pad - 1, jnp.int32)])
    dst_w = dst_p.reshape(NW, nch, CH)

    vec_cm = vec_feat.transpose(0, 2, 1).reshape(n, 3 * v)
    coordt = jnp.pad(coord_feat, ((0, node_pad - n), (0, 13)))
    aux = jnp.concatenate(
        [vec_cm, jnp.pad(coord_feat, ((0, 0), (0, 13)))], axis=1)

    rows, aux_g, coordd = _sc_gather(scalar_feat, aux, coordt, src_w, dst_w)
    msg = _edge_tc(rows, aux_g, coordd, msg_params, s, v)

    nch2 = e_pad // (NUM_SUBCORES * CH)
    dst_t = dst_p.reshape(NUM_SUBCORES, nch2, CH)
    zeros_hbm = jnp.zeros((node_pad, 128), F32)
    parts = _sc_scatter(msg, dst_t, zeros_hbm)

    sca_pad = jnp.pad(scalar_feat, ((0, node_pad - n), (0, 0)))
    vec_pad = jnp.pad(vec_cm, ((0, node_pad - n), (0, 0)))
    s_full, v_full = _node_tc(sca_pad, vec_pad, parts, upd_params,
                              ln1_g, ln1_b, ln2_g, ln2_b, s, v)

    s_out = s_full[:n]
    v_out = v_full[:n].reshape(n, 3, v).transpose(0, 2, 1)
    return (s_out, v_out)
